# Initial kernel scaffold; baseline (speedup 1.0000x reference)
import jax
import jax.numpy as jnp
from jax.experimental import pallas as pl


def kernel(x, W_enc, b_enc, W_dec, b_dec):
    raise NotImplementedError("write your pallas kernel here")


# speedup vs baseline: 9.3589x; 9.3589x over previous
import functools

import jax
import jax.numpy as jnp
from jax.experimental import pallas as pl

_K = 32


def _enc_body(x_ref, w_ref, b_ref, out_ref):
    acc = jax.lax.dot_general(
        x_ref[...], w_ref[...],
        dimension_numbers=(((1,), (1,)), ((), ())),
        preferred_element_type=jnp.float32,
    )
    out_ref[...] = acc + b_ref[...]


def _topk_body(h_ref, out_ref):
    h = h_ref[...]
    bm = h.shape[0]
    b = jax.lax.bitcast_convert_type(h, jnp.int32)
    s = jnp.where(b < 0, b ^ jnp.int32(0x7FFFFFFF), b)

    def bit_step(i, t):
        bit = 31 - i
        cand = t + (jnp.int32(1) << bit)
        cnt = jnp.sum((s >= cand[:, None]).astype(jnp.int32), axis=1)
        return jnp.where(cnt >= _K, cand, t)

    t0 = jnp.full((bm,), jnp.int32(-2147483647) - 1)
    t = jax.lax.fori_loop(0, 32, bit_step, t0)
    out_ref[...] = jnp.where(s >= t[:, None], h, 0.0)


def _dec_body(hs_ref, w_ref, b_ref, out_ref):
    k = pl.program_id(1)

    @pl.when(k == 0)
    def _():
        out_ref[...] = jnp.broadcast_to(b_ref[...], out_ref.shape)

    out_ref[...] += jax.lax.dot_general(
        hs_ref[...], w_ref[...],
        dimension_numbers=(((1,), (1,)), ((), ())),
        preferred_element_type=jnp.float32,
    )


@jax.jit
def kernel(x, W_enc, b_enc, W_dec, b_dec):
    B, D = x.shape
    H = W_enc.shape[0]

    bm = min(1024, B)
    bn = min(1024, H)
    h = pl.pallas_call(
        _enc_body,
        grid=(B // bm, H // bn),
        in_specs=[
            pl.BlockSpec((bm, D), lambda i, j: (i, 0)),
            pl.BlockSpec((bn, D), lambda i, j: (j, 0)),
            pl.BlockSpec((1, bn), lambda i, j: (0, j)),
        ],
        out_specs=pl.BlockSpec((bm, bn), lambda i, j: (i, j)),
        out_shape=jax.ShapeDtypeStruct((B, H), jnp.float32),
    )(x, W_enc, b_enc.reshape(1, H))

    bt = min(256, B)
    h_sparse = pl.pallas_call(
        _topk_body,
        grid=(B // bt,),
        in_specs=[pl.BlockSpec((bt, H), lambda i: (i, 0))],
        out_specs=pl.BlockSpec((bt, H), lambda i: (i, 0)),
        out_shape=jax.ShapeDtypeStruct((B, H), jnp.float32),
    )(h)

    bk = min(1024, H)
    recon = pl.pallas_call(
        _dec_body,
        grid=(B // bm, H // bk),
        in_specs=[
            pl.BlockSpec((bm, bk), lambda i, k: (i, k)),
            pl.BlockSpec((D, bk), lambda i, k: (0, k)),
            pl.BlockSpec((1, D), lambda i, k: (0, 0)),
        ],
        out_specs=pl.BlockSpec((bm, D), lambda i, k: (i, 0)),
        out_shape=jax.ShapeDtypeStruct((B, D), jnp.float32),
    )(h_sparse, W_dec, b_dec.reshape(1, D))

    return (h_sparse, recon)
